# baseline (speedup 1.0000x reference)
import functools
import jax
import jax.numpy as jnp
from jax import lax
from jax.experimental import pallas as pl
from jax.experimental.pallas import tpu as pltpu
from jax.experimental.pallas import tpu_sc as plsc

D_MODEL = 64
SCALE = 8.0
CHUNK = 128
NC, NS, L = 2, 16, 16
NW = NC * NS


@jax.jit
def kernel(x, lut):
    b0, b1 = x.shape
    n = b0 * b1
    assert n % (NW * CHUNK) == 0
    n_chunks = n // CHUNK
    chunks_per_w = n_chunks // NW
    idx = x.reshape(n_chunks, CHUNK).astype(jnp.int32)

    mesh = plsc.VectorSubcoreMesh(core_axis_name="c", subcore_axis_name="s")

    @functools.partial(
        pl.kernel,
        out_type=jax.ShapeDtypeStruct((n, D_MODEL), jnp.float32),
        mesh=mesh,
        compiler_params=pltpu.CompilerParams(use_tc_tiling_on_sc=False),
        scratch_types=[
            pltpu.VMEM((chunks_per_w, CHUNK), jnp.int32),
            pltpu.VMEM((CHUNK, D_MODEL), jnp.float32),
            pltpu.SemaphoreType.DMA,
        ],
    )
    def run(lut_hbm, idx_hbm, out_hbm, idx_v, rows_v, sem):
        wid = lax.axis_index("c") * NS + lax.axis_index("s")
        base = wid * chunks_per_w
        pltpu.sync_copy(idx_hbm.at[pl.ds(base, chunks_per_w)], idx_v)

        @pl.loop(0, chunks_per_w)
        def _chunk(j):
            pltpu.async_copy(lut_hbm.at[idx_v.at[j]], rows_v, sem).wait()

            @pl.loop(0, CHUNK)
            def _row(r):
                for c in range(D_MODEL // L):
                    sl = rows_v.at[r, pl.ds(c * L, L)]
                    sl[...] = sl[...] * SCALE

            pltpu.sync_copy(
                rows_v, out_hbm.at[pl.ds((base + j) * CHUNK, CHUNK)]
            )

    out = run(lut, idx)
    return out.reshape(b0, b1, D_MODEL)
